# baseline (speedup 1.0000x reference)
import functools

import jax
import jax.numpy as jnp
from jax import lax
from jax.experimental import pallas as pl
from jax.experimental.pallas import tpu as pltpu
from jax.experimental.pallas import tpu_sc as plsc

C = 100
K = 4
NW = 32
LANES = 16


def _sc_embed(idx2, token_table, pos3):
    nchunk, c = idx2.shape
    vocab, d = token_table.shape
    chunks_per_w = nchunk // NW
    nblk = chunks_per_w // K
    assert c == C and nchunk % (NW * K) == 0 and d % LANES == 0

    mesh = plsc.VectorSubcoreMesh(core_axis_name="c", subcore_axis_name="s")

    @functools.partial(
        pl.kernel,
        mesh=mesh,
        out_type=jax.ShapeDtypeStruct((nchunk, C, d), jnp.float32),
        compiler_params=pltpu.CompilerParams(use_tc_tiling_on_sc=False),
        scratch_types=[
            pltpu.VMEM((2, C, d), jnp.float32),
            pltpu.VMEM((K, C), jnp.int32),
            pltpu.VMEM((K, C, d), jnp.float32),
            pltpu.SemaphoreType.DMA,
        ],
    )
    def body(idx_hbm, tab_hbm, pos_hbm, out_hbm, pos_v, idx_v, rows_v, gsem):
        wid = lax.axis_index("s") * 2 + lax.axis_index("c")
        base = wid * chunks_per_w
        pltpu.sync_copy(pos_hbm, pos_v)

        def blk_body(blk, carry):
            c0 = base + blk * K
            pltpu.sync_copy(idx_hbm.at[pl.ds(c0, K)], idx_v)
            for j in range(K):
                pltpu.async_copy(tab_hbm.at[idx_v.at[j]], rows_v.at[j], gsem).wait()

            def radd(r, rc):
                for j in range(K):
                    for v in range(d // LANES):
                        s = pl.ds(v * LANES, LANES)
                        rows_v[j, r, s] = rows_v[j, r, s] + pos_v[j % 2, r, s]
                return rc

            lax.fori_loop(0, C, radd, 0)
            pltpu.sync_copy(rows_v, out_hbm.at[pl.ds(c0, K)])
            return carry

        lax.fori_loop(0, nblk, blk_body, 0)

    return body(idx2, token_table, pos3)


def kernel(inputs, token_table, pos_table):
    b, s = inputs.shape
    _, d = token_table.shape
    idx2 = inputs.astype(jnp.int32).reshape(-1, C)
    pos3 = pos_table.reshape(s // C, C, d)
    out = _sc_embed(idx2, token_table, pos3)
    return out.reshape(b, s, d)


# speedup vs baseline: 1.2410x; 1.2410x over previous
import functools

import jax
import jax.numpy as jnp
from jax import lax
from jax.experimental import pallas as pl
from jax.experimental.pallas import tpu as pltpu
from jax.experimental.pallas import tpu_sc as plsc

C = 100
K = 2
NBUF = 4
NW = 32
LANES = 16


def _sc_embed(idx2, token_table, pos3):
    nchunk, c = idx2.shape
    vocab, d = token_table.shape
    cpw = nchunk // NW
    nblk = cpw // K
    assert c == C and nchunk % (NW * K) == 0 and d % LANES == 0
    assert nblk > NBUF

    mesh = plsc.VectorSubcoreMesh(core_axis_name="c", subcore_axis_name="s")

    @functools.partial(
        pl.kernel,
        mesh=mesh,
        out_type=jax.ShapeDtypeStruct((nchunk, C, d), jnp.float32),
        compiler_params=pltpu.CompilerParams(use_tc_tiling_on_sc=False),
        scratch_types=[
            pltpu.VMEM((2, C, d), jnp.float32),
            pltpu.VMEM((cpw, C), jnp.int32),
            pltpu.VMEM((NBUF, K, C, d), jnp.float32),
        ]
        + [pltpu.SemaphoreType.DMA] * (2 * NBUF),
    )
    def body(idx_hbm, tab_hbm, pos_hbm, out_hbm, pos_v, idx_v, rows_v, *sems):
        gsem = sems[:NBUF]
        ssem = sems[NBUF:]
        wid = lax.axis_index("s") * 2 + lax.axis_index("c")
        base_c = wid * cpw
        pltpu.sync_copy(pos_hbm, pos_v)
        pltpu.sync_copy(idx_hbm.at[pl.ds(base_c, cpw)], idx_v)

        def fire_gathers(g, b):
            for j in range(K):
                pltpu.async_copy(
                    tab_hbm.at[idx_v.at[g * K + j]], rows_v.at[b, j], gsem[b]
                )

        for g0 in range(NBUF - 1):
            fire_gathers(g0, g0)

        def phase(i, b):
            g = i * NBUF + b
            pltpu.make_async_copy(
                out_hbm.at[pl.ds(0, K)], rows_v.at[b], gsem[b]
            ).wait()

            def radd(r, rc):
                for j in range(K):
                    for v in range(d // LANES):
                        s = pl.ds(v * LANES, LANES)
                        rows_v[b, j, r, s] = rows_v[b, j, r, s] + pos_v[j % 2, r, s]
                return rc

            lax.fori_loop(0, C, radd, 0)
            pltpu.async_copy(
                rows_v.at[b], out_hbm.at[pl.ds(base_c + g * K, K)], ssem[b]
            )
            g2 = g + NBUF - 1
            b2 = (b + NBUF - 1) % NBUF

            @pl.when(g2 < nblk)
            def _():
                @pl.when(g >= 1)
                def _():
                    pltpu.make_async_copy(
                        rows_v.at[b2], out_hbm.at[pl.ds(0, K)], ssem[b2]
                    ).wait()

                fire_gathers(g2, b2)

            return 0

        def blk_cycle(i, carry):
            for b in range(NBUF):
                phase(i, b)
            return carry

        lax.fori_loop(0, nblk // NBUF, blk_cycle, 0)
        for b in range(NBUF):
            pltpu.make_async_copy(
                rows_v.at[b], out_hbm.at[pl.ds(0, K)], ssem[b]
            ).wait()

    return body(idx2, token_table, pos3)


def kernel(inputs, token_table, pos_table):
    b, s = inputs.shape
    _, d = token_table.shape
    idx2 = inputs.astype(jnp.int32).reshape(-1, C)
    pos3 = pos_table.reshape(s // C, C, d)
    out = _sc_embed(idx2, token_table, pos3)
    return out.reshape(b, s, d)


# speedup vs baseline: 1.2437x; 1.0021x over previous
import functools

import jax
import jax.numpy as jnp
from jax import lax
from jax.experimental import pallas as pl
from jax.experimental.pallas import tpu as pltpu
from jax.experimental.pallas import tpu_sc as plsc

GSIZES = (128, 72)
NBUF = 4
NW = 32
LANES = 16


def _sc_embed(idx, token_table, pos_table):
    nseq, s = idx.shape
    vocab, d = token_table.shape
    spw = nseq // NW
    assert s == sum(GSIZES) and nseq % NW == 0 and d % LANES == 0
    assert spw > NBUF

    mesh = plsc.VectorSubcoreMesh(core_axis_name="c", subcore_axis_name="s")

    @functools.partial(
        pl.kernel,
        mesh=mesh,
        out_type=jax.ShapeDtypeStruct((nseq, s, d), jnp.float32),
        compiler_params=pltpu.CompilerParams(use_tc_tiling_on_sc=False),
        scratch_types=[
            pltpu.VMEM((s, d), jnp.float32),
            pltpu.VMEM((spw, s), jnp.int32),
            pltpu.VMEM((NBUF, s, d), jnp.float32),
        ]
        + [pltpu.SemaphoreType.DMA] * (2 * NBUF),
    )
    def body(idx_hbm, tab_hbm, pos_hbm, out_hbm, pos_v, idx_v, rows_v, *sems):
        gsem = sems[:NBUF]
        ssem = sems[NBUF:]
        wid = lax.axis_index("s") * 2 + lax.axis_index("c")
        base = wid * spw
        pltpu.sync_copy(pos_hbm, pos_v)
        pltpu.sync_copy(idx_hbm.at[pl.ds(base, spw)], idx_v)

        def fire_gathers(g, b):
            off = 0
            for n in GSIZES:
                pltpu.async_copy(
                    tab_hbm.at[idx_v.at[g, pl.ds(off, n)]],
                    rows_v.at[b, pl.ds(off, n)],
                    gsem[b],
                )
                off += n

        for g0 in range(NBUF - 1):
            fire_gathers(g0, g0)

        def phase(i, b):
            g = i * NBUF + b
            pltpu.make_async_copy(out_hbm.at[0], rows_v.at[b], gsem[b]).wait()

            def radd(r, rc):
                for v in range(d // LANES):
                    vs = pl.ds(v * LANES, LANES)
                    rows_v[b, r, vs] = rows_v[b, r, vs] + pos_v[r, vs]
                return rc

            lax.fori_loop(0, s, radd, 0)
            pltpu.async_copy(rows_v.at[b], out_hbm.at[base + g], ssem[b])
            g2 = g + NBUF - 1
            b2 = (b + NBUF - 1) % NBUF

            @pl.when(g2 < spw)
            def _():
                @pl.when(g >= 1)
                def _():
                    pltpu.make_async_copy(
                        rows_v.at[b2], out_hbm.at[0], ssem[b2]
                    ).wait()

                fire_gathers(g2, b2)

            return 0

        def blk_cycle(i, carry):
            for b in range(NBUF):
                phase(i, b)
            return carry

        lax.fori_loop(0, spw // NBUF, blk_cycle, 0)
        for b in range(NBUF):
            pltpu.make_async_copy(rows_v.at[b], out_hbm.at[0], ssem[b]).wait()

    return body(idx, token_table, pos_table)


def kernel(inputs, token_table, pos_table):
    return _sc_embed(inputs.astype(jnp.int32), token_table, pos_table)
